# baseline (speedup 1.0000x reference)
import jax
import jax.numpy as jnp
from jax import lax
from jax.experimental import pallas as pl
from jax.experimental.pallas import tpu as pltpu
from jax.experimental.pallas import tpu_sc as plsc

_F = 26
_V = 100000
_D = 16
_B = 16384

_NC = 2
_NS = 16
_NW = _NC * _NS
_NR = _F * _D
_RPW = _NR // _NW
_QB = _B // 4


def _body(x_hbm, tab_hbm, out_hbm, trow_v, xcol_v, ob_a, ob_b, osem_a, osem_b):
    wid = lax.axis_index("s") * _NC + lax.axis_index("c")

    obufs = (ob_a, ob_b)
    osems = (osem_a, osem_b)

    def out_slice(r, q):
        return out_hbm.at[r, pl.ds(q * _QB, _QB)]

    f_prev = None
    for k in range(_RPW):
        r = wid * _RPW + k
        f = lax.div(r, _D)
        d = lax.rem(r, _D)
        pltpu.sync_copy(tab_hbm.at[f, d], trow_v)
        if k == 0:
            pltpu.sync_copy(x_hbm.at[f], xcol_v)
        else:
            @pl.when(f != f_prev)
            def _():
                pltpu.sync_copy(x_hbm.at[f], xcol_v)
        f_prev = f
        for q in range(4):
            obuf, osem = obufs[q % 2], osems[q % 2]
            if 4 * k + q >= 2:
                pltpu.make_async_copy(obuf, out_slice(r, q), osem).wait()

            def extract(j, carry):
                idx16 = xcol_v[pl.ds(q * _QB + j * 16, 16)]
                obuf[pl.ds(j * 16, 16)] = plsc.load_gather(trow_v, [idx16])
                return carry

            lax.fori_loop(0, _QB // 16, extract, 0, unroll=1)
            pltpu.async_copy(obuf, out_slice(r, q), osem)
    for q in (2, 3):
        r = wid * _RPW + _RPW - 1
        pltpu.make_async_copy(obufs[q % 2], out_slice(r, q), osems[q % 2]).wait()


@jax.jit
def _mce(xT, tab_t):
    mesh = plsc.VectorSubcoreMesh(core_axis_name="c", subcore_axis_name="s",
                                  num_cores=_NC, num_subcores=_NS)
    return pl.kernel(
        _body,
        out_type=jax.ShapeDtypeStruct((_NR, _B), jnp.float32),
        mesh=mesh,
        scratch_types=[
            pltpu.VMEM((_V,), jnp.float32),
            pltpu.VMEM((_B,), jnp.int32),
            pltpu.VMEM((_QB,), jnp.float32),
            pltpu.VMEM((_QB,), jnp.float32),
            pltpu.SemaphoreType.DMA,
            pltpu.SemaphoreType.DMA,
        ],
        compiler_params=pltpu.CompilerParams(use_tc_tiling_on_sc=True,
                                             needs_layout_passes=False),
    )(xT, tab_t)


def kernel(X, tables):
    tab_t = jnp.transpose(tables, (0, 2, 1))
    xT = X.astype(jnp.int32).T
    res = _mce(xT, tab_t)
    return res.T


# speedup vs baseline: 1.0331x; 1.0331x over previous
import jax
import jax.numpy as jnp
from jax import lax
from jax.experimental import pallas as pl
from jax.experimental.pallas import tpu as pltpu
from jax.experimental.pallas import tpu_sc as plsc

_F = 26
_V = 100000
_D = 16
_B = 16384

_NC = 2
_NS = 16
_NW = _NC * _NS
_NR = _F * _D
_RPW = _NR // _NW
_QB = _B // 4


def _body(x_hbm, tab_hbm, out_hbm, trow_v, xcol_v, ob_a, ob_b, osem_a, osem_b):
    wid = lax.axis_index("s") * _NC + lax.axis_index("c")

    obufs = (ob_a, ob_b)
    osems = (osem_a, osem_b)

    def out_slice(r, q):
        return out_hbm.at[r, pl.ds(q * _QB, _QB)]

    f_prev = None
    for k in range(_RPW):
        r = wid * _RPW + k
        f = lax.div(r, _D)
        d = lax.rem(r, _D)
        pltpu.sync_copy(tab_hbm.at[f, d], trow_v)
        if k == 0:
            pltpu.sync_copy(x_hbm.at[f], xcol_v)
        else:
            @pl.when(f != f_prev)
            def _():
                pltpu.sync_copy(x_hbm.at[f], xcol_v)
        f_prev = f
        for q in range(4):
            obuf, osem = obufs[q % 2], osems[q % 2]
            if 4 * k + q >= 2:
                pltpu.make_async_copy(obuf, out_slice(r, q), osem).wait()

            def extract(j, carry):
                for l in range(4):
                    idx16 = xcol_v[pl.ds(q * _QB + j * 64 + l * 16, 16)]
                    obuf[pl.ds(j * 64 + l * 16, 16)] = plsc.load_gather(
                        trow_v, [idx16])
                return carry

            lax.fori_loop(0, 64, extract, 0, unroll=1)
            pltpu.async_copy(obuf, out_slice(r, q), osem)
    for q in (2, 3):
        r = wid * _RPW + _RPW - 1
        pltpu.make_async_copy(obufs[q % 2], out_slice(r, q), osems[q % 2]).wait()


@jax.jit
def _mce(xT, tab_t):
    mesh = plsc.VectorSubcoreMesh(core_axis_name="c", subcore_axis_name="s",
                                  num_cores=_NC, num_subcores=_NS)
    return pl.kernel(
        _body,
        out_type=jax.ShapeDtypeStruct((_NR, _B), jnp.float32),
        mesh=mesh,
        scratch_types=[
            pltpu.VMEM((_V,), jnp.float32),
            pltpu.VMEM((_B,), jnp.int32),
            pltpu.VMEM((_QB,), jnp.float32),
            pltpu.VMEM((_QB,), jnp.float32),
            pltpu.SemaphoreType.DMA,
            pltpu.SemaphoreType.DMA,
        ],
        compiler_params=pltpu.CompilerParams(use_tc_tiling_on_sc=True,
                                             needs_layout_passes=False),
    )(xT, tab_t)


def kernel(X, tables):
    tab_t = jnp.transpose(tables, (0, 2, 1))
    xT = X.astype(jnp.int32).T
    res = _mce(xT, tab_t)
    return res.T


# speedup vs baseline: 1.3359x; 1.2932x over previous
import jax
import jax.numpy as jnp
from jax import lax
from jax.experimental import pallas as pl
from jax.experimental.pallas import tpu as pltpu
from jax.experimental.pallas import tpu_sc as plsc

_F = 26
_V = 100000
_D = 16
_B = 16384

_NC = 2
_NS = 16
_NW = _NC * _NS
_NR = _F * _D
_RPW = _NR // _NW
_QB = _B // 4


def _body(x_hbm, tab_hbm, out_hbm, trow_v, xcol_v, ob_a, ob_b, osem_a, osem_b):
    wid = lax.axis_index("s") * _NC + lax.axis_index("c")

    obufs = (ob_a, ob_b)
    osems = (osem_a, osem_b)

    def out_slice(r, q):
        return out_hbm.at[r, pl.ds(q * _QB, _QB)]

    f_prev = None
    for k in range(_RPW):
        r = wid * _RPW + k
        f = lax.div(r, _D)
        d = lax.rem(r, _D)
        pltpu.sync_copy(tab_hbm.at[f, d], trow_v)
        if k == 0:
            pltpu.sync_copy(x_hbm.at[f], xcol_v)
        else:
            @pl.when(f != f_prev)
            def _():
                pltpu.sync_copy(x_hbm.at[f], xcol_v)
        f_prev = f
        for q in range(4):
            obuf, osem = obufs[q % 2], osems[q % 2]
            if 4 * k + q >= 2:
                pltpu.make_async_copy(obuf, out_slice(r, q), osem).wait()

            def extract(j, carry):
                for l in range(8):
                    idx16 = xcol_v[pl.ds(q * _QB + j * 128 + l * 16, 16)]
                    obuf[pl.ds(j * 128 + l * 16, 16)] = plsc.load_gather(
                        trow_v, [idx16])
                return carry

            lax.fori_loop(0, 32, extract, 0, unroll=1)
            pltpu.async_copy(obuf, out_slice(r, q), osem)
    for q in (2, 3):
        r = wid * _RPW + _RPW - 1
        pltpu.make_async_copy(obufs[q % 2], out_slice(r, q), osems[q % 2]).wait()


@jax.jit
def _mce(xT, tab_t):
    mesh = plsc.VectorSubcoreMesh(core_axis_name="c", subcore_axis_name="s",
                                  num_cores=_NC, num_subcores=_NS)
    return pl.kernel(
        _body,
        out_type=jax.ShapeDtypeStruct((_NR, _B), jnp.float32),
        mesh=mesh,
        scratch_types=[
            pltpu.VMEM((_V,), jnp.float32),
            pltpu.VMEM((_B,), jnp.int32),
            pltpu.VMEM((_QB,), jnp.float32),
            pltpu.VMEM((_QB,), jnp.float32),
            pltpu.SemaphoreType.DMA,
            pltpu.SemaphoreType.DMA,
        ],
        compiler_params=pltpu.CompilerParams(use_tc_tiling_on_sc=True,
                                             needs_layout_passes=False),
    )(xT, tab_t)


def kernel(X, tables):
    tab_t = jnp.transpose(tables, (0, 2, 1))
    xT = X.astype(jnp.int32).T
    res = _mce(xT, tab_t)
    return res.T
